# initial kernel scaffold (unmeasured)
import jax
import jax.numpy as jnp
from jax import lax
from jax.experimental import pallas as pl
from jax.experimental.pallas import tpu as pltpu


def kernel(
    x,
):
    def body(*refs):
        pass

    out_shape = jax.ShapeDtypeStruct(..., jnp.float32)
    return pl.pallas_call(body, out_shape=out_shape)(...)


# baseline (device time: 31282 ns/iter reference)
import jax
import jax.numpy as jnp
from jax import lax
from jax.experimental import pallas as pl
from jax.experimental.pallas import tpu as pltpu

M = 512
NCOL = 512


def kernel(x):
    def body(x_ref, out_ref, comm_ref, red_ref, send_sems, recv_sems):
        my_x = lax.axis_index("x")
        my_y = lax.axis_index("y")
        x_peer = (1 - my_x, my_y)
        y_peer = (my_x, 1 - my_y)

        barrier_sem = pltpu.get_barrier_semaphore()
        for peer in (x_peer, y_peer):
            pl.semaphore_signal(
                barrier_sem, inc=1,
                device_id=peer, device_id_type=pl.DeviceIdType.MESH,
            )
        pl.semaphore_wait(barrier_sem, 2)

        rdma1 = pltpu.make_async_remote_copy(
            src_ref=x_ref.at[0],
            dst_ref=comm_ref.at[0],
            send_sem=send_sems.at[0],
            recv_sem=recv_sems.at[0],
            device_id=x_peer,
            device_id_type=pl.DeviceIdType.MESH,
        )
        rdma1.start()
        rdma1.wait()

        red = x_ref[0] + comm_ref[0]
        out_ref[:, pl.ds(my_y * NCOL, NCOL)] = red
        red_ref[...] = red

        rdma2 = pltpu.make_async_remote_copy(
            src_ref=red_ref,
            dst_ref=comm_ref.at[1],
            send_sem=send_sems.at[1],
            recv_sem=recv_sems.at[1],
            device_id=y_peer,
            device_id_type=pl.DeviceIdType.MESH,
        )
        rdma2.start()
        rdma2.wait()

        out_ref[:, pl.ds((1 - my_y) * NCOL, NCOL)] = comm_ref[1]

    return pl.pallas_call(
        body,
        out_shape=jax.ShapeDtypeStruct((M, 2 * NCOL), jnp.float32),
        in_specs=[pl.BlockSpec(memory_space=pltpu.VMEM)],
        out_specs=pl.BlockSpec(memory_space=pltpu.VMEM),
        scratch_shapes=[
            pltpu.VMEM((2, M, NCOL), jnp.float32),
            pltpu.VMEM((M, NCOL), jnp.float32),
            pltpu.SemaphoreType.DMA((2,)),
            pltpu.SemaphoreType.DMA((2,)),
        ],
        compiler_params=pltpu.CompilerParams(collective_id=0),
    )(x)


# device time: 21495 ns/iter; 1.4553x vs baseline; 1.4553x over previous
import jax
import jax.numpy as jnp
from jax import lax
from jax.experimental import pallas as pl
from jax.experimental.pallas import tpu as pltpu

M = 512
NCOL = 512
C = 8
RC = M // C


def kernel(x):
    def body(x_ref, out_ref, comm_ref, send1, recv1, send2, recv2):
        my_x = lax.axis_index("x")
        my_y = lax.axis_index("y")
        x_peer = (1 - my_x, my_y)
        y_peer = (my_x, 1 - my_y)
        my_col = my_y * NCOL

        barrier_sem = pltpu.get_barrier_semaphore()
        for peer in (x_peer, y_peer):
            pl.semaphore_signal(
                barrier_sem, inc=1,
                device_id=peer, device_id_type=pl.DeviceIdType.MESH,
            )
        pl.semaphore_wait(barrier_sem, 2)

        rdma1 = []
        for c in range(C):
            rows = pl.ds(c * RC, RC)
            r = pltpu.make_async_remote_copy(
                src_ref=x_ref.at[0, rows, :],
                dst_ref=comm_ref.at[rows, :],
                send_sem=send1.at[c],
                recv_sem=recv1.at[c],
                device_id=x_peer,
                device_id_type=pl.DeviceIdType.MESH,
            )
            r.start()
            rdma1.append(r)

        rdma2 = []
        for c in range(C):
            rows = pl.ds(c * RC, RC)
            rdma1[c].wait_recv()
            out_ref[rows, pl.ds(my_col, NCOL)] = (
                x_ref[0, rows, :] + comm_ref[rows, :]
            )
            r = pltpu.make_async_remote_copy(
                src_ref=out_ref.at[rows, pl.ds(my_col, NCOL)],
                dst_ref=out_ref.at[rows, pl.ds(my_col, NCOL)],
                send_sem=send2.at[c],
                recv_sem=recv2.at[c],
                device_id=y_peer,
                device_id_type=pl.DeviceIdType.MESH,
            )
            r.start()
            rdma2.append(r)

        for c in range(C):
            rdma1[c].wait_send()
            rdma2[c].wait_recv()
            rdma2[c].wait_send()

    return pl.pallas_call(
        body,
        out_shape=jax.ShapeDtypeStruct((M, 2 * NCOL), jnp.float32),
        in_specs=[pl.BlockSpec(memory_space=pltpu.VMEM)],
        out_specs=pl.BlockSpec(memory_space=pltpu.VMEM),
        scratch_shapes=[
            pltpu.VMEM((M, NCOL), jnp.float32),
            pltpu.SemaphoreType.DMA((C,)),
            pltpu.SemaphoreType.DMA((C,)),
            pltpu.SemaphoreType.DMA((C,)),
            pltpu.SemaphoreType.DMA((C,)),
        ],
        compiler_params=pltpu.CompilerParams(collective_id=0),
    )(x)


# device time: 21077 ns/iter; 1.4842x vs baseline; 1.0198x over previous
import jax
import jax.numpy as jnp
from jax import lax
from jax.experimental import pallas as pl
from jax.experimental.pallas import tpu as pltpu

M = 512
NCOL = 512
C = 16
RC = M // C


def kernel(x):
    def body(x_ref, out_ref, comm_ref, send1, recv1, send2, recv2):
        my_x = lax.axis_index("x")
        my_y = lax.axis_index("y")
        x_peer = (1 - my_x, my_y)
        y_peer = (my_x, 1 - my_y)
        my_col = my_y * NCOL

        barrier_sem = pltpu.get_barrier_semaphore()
        for peer in (x_peer, y_peer):
            pl.semaphore_signal(
                barrier_sem, inc=1,
                device_id=peer, device_id_type=pl.DeviceIdType.MESH,
            )
        pl.semaphore_wait(barrier_sem, 2)

        rdma1 = []
        for c in range(C):
            rows = pl.ds(c * RC, RC)
            r = pltpu.make_async_remote_copy(
                src_ref=x_ref.at[0, rows, :],
                dst_ref=comm_ref.at[rows, :],
                send_sem=send1.at[c],
                recv_sem=recv1.at[c],
                device_id=x_peer,
                device_id_type=pl.DeviceIdType.MESH,
            )
            r.start()
            rdma1.append(r)

        rdma2 = []
        for c in range(C):
            rows = pl.ds(c * RC, RC)
            rdma1[c].wait_recv()
            out_ref[rows, pl.ds(my_col, NCOL)] = (
                x_ref[0, rows, :] + comm_ref[rows, :]
            )
            r = pltpu.make_async_remote_copy(
                src_ref=out_ref.at[rows, pl.ds(my_col, NCOL)],
                dst_ref=out_ref.at[rows, pl.ds(my_col, NCOL)],
                send_sem=send2.at[c],
                recv_sem=recv2.at[c],
                device_id=y_peer,
                device_id_type=pl.DeviceIdType.MESH,
            )
            r.start()
            rdma2.append(r)

        for c in range(C):
            rdma1[c].wait_send()
            rdma2[c].wait_recv()
            rdma2[c].wait_send()

    return pl.pallas_call(
        body,
        out_shape=jax.ShapeDtypeStruct((M, 2 * NCOL), jnp.float32),
        in_specs=[pl.BlockSpec(memory_space=pltpu.VMEM)],
        out_specs=pl.BlockSpec(memory_space=pltpu.VMEM),
        scratch_shapes=[
            pltpu.VMEM((M, NCOL), jnp.float32),
            pltpu.SemaphoreType.DMA((C,)),
            pltpu.SemaphoreType.DMA((C,)),
            pltpu.SemaphoreType.DMA((C,)),
            pltpu.SemaphoreType.DMA((C,)),
        ],
        compiler_params=pltpu.CompilerParams(collective_id=0),
    )(x)
